# Initial kernel scaffold; baseline (speedup 1.0000x reference)
import jax
import jax.numpy as jnp
from jax.experimental import pallas as pl


def kernel(x, patch_fc1_w, patch_fc1_b, patch_fc2_w, patch_fc2_b, gate_pair, atom_in_w, atom_in_b, atom_out_w, atom_out_b):
    raise NotImplementedError("write your pallas kernel here")


# speedup vs baseline: 1.5753x; 1.5753x over previous
import functools

import jax
import jax.numpy as jnp
import numpy as np
from jax.experimental import pallas as pl
from jax.experimental.pallas import tpu as pltpu
from jax.sharding import Mesh, PartitionSpec as P

IN_DIM = 768
HIDDEN = 3072
NCLS = 6
B = 32
NPATCH = 576
NTOK = NCLS + NPATCH

SRC_TBL = np.array([[0, 3], [0, 4], [1, 3], [1, 4], [2, 3], [2, 4]], dtype=np.int32)
DST_TBL = np.array([[3, 0], [4, 0], [3, 1], [4, 1], [3, 2], [4, 2]], dtype=np.int32)

SRC_BY_ATOM = [[] for _ in range(5)]
DST_BY_ATOM = [[] for _ in range(5)]
for _n in range(NCLS):
    for _r in range(2):
        SRC_BY_ATOM[SRC_TBL[_n, _r]].append(_n * 2 + _r)
        DST_BY_ATOM[DST_TBL[_n, _r]].append(_n * 2 + _r)

IN_ATOM = [0, 1, 2, 3, 4, 4, 4, 4]
OUT_ATOM = [3, 3, 3, 3, 4, 0, 1, 2]
CAND_STEP_ATOM = {3: 3, 4: 4, 5: 0, 6: 1, 7: 2}


def _gelu(v):
    return v * (0.5 * jax.lax.erf(v * np.float32(1.0 / np.sqrt(2.0))) + 0.5)


def _cls_kernel(bl, cls_ref, gate_ref, in_w_ref, in_b_ref, out_w_ref,
                out_b_ref, out_ref, hp_ref, gsel_ref):
    s = pl.program_id(0)

    @pl.when(s == 0)
    def _gate():
        cls = cls_ref[...]
        gw = gate_ref[...]
        inv_cn = jax.lax.rsqrt(jnp.maximum(jnp.sum(cls * cls, axis=-1), 1e-24))
        inv_gn = jax.lax.rsqrt(jnp.maximum(jnp.sum(gw * gw, axis=-1), 1e-24))
        l0 = jnp.sum(cls * gw[:, 0, :][None], axis=-1) * inv_cn * inv_gn[None, :, 0]
        l1 = jnp.sum(cls * gw[:, 1, :][None], axis=-1) * inv_cn * inv_gn[None, :, 1]
        p0 = jax.nn.sigmoid(l0 - l1)
        pick0 = l0 >= l1
        gsel_ref[0, :, :] = jnp.where(pick0, p0, 0.0)
        gsel_ref[1, :, :] = jnp.where(pick0, 0.0, 1.0 - p0)

    @pl.when(s < 5)
    def _hidden():
        in_w = in_w_ref[0].astype(jnp.bfloat16)
        in_b = in_b_ref[0]
        for a in range(5):
            @pl.when(s == a)
            def _do(a=a):
                ks = SRC_BY_ATOM[a]
                toks = jnp.concatenate(
                    [cls_ref[:, k // 2, :] for k in ks], axis=0
                ).astype(jnp.bfloat16)
                h = jax.lax.dot_general(
                    toks, in_w, (((1,), (1,)), ((), ())),
                    preferred_element_type=jnp.float32)
                h = _gelu(h + in_b).astype(jnp.bfloat16)
                for i, k in enumerate(ks):
                    hp_ref[k * bl:(k + 1) * bl, :] = h[i * bl:(i + 1) * bl, :]

    @pl.when(s >= 3)
    def _cand():
        out_w = out_w_ref[0].astype(jnp.bfloat16)
        out_b = out_b_ref[0]
        for step, d in CAND_STEP_ATOM.items():
            @pl.when(s == step)
            def _do(step=step, d=d):
                ks = DST_BY_ATOM[d]
                h = jnp.concatenate(
                    [hp_ref[k * bl:(k + 1) * bl, :] for k in ks], axis=0)
                cand = jax.lax.dot_general(
                    h, out_w, (((1,), (1,)), ((), ())),
                    preferred_element_type=jnp.float32) + out_b
                for i, k in enumerate(ks):
                    n, r = k // 2, k % 2
                    w = gsel_ref[r, :, n].reshape(bl, 1)
                    piece = w * cand[i * bl:(i + 1) * bl, :]
                    if r == 0:
                        out_ref[:, n, :] = piece
                    else:
                        out_ref[:, n, :] = out_ref[:, n, :] + piece


def _patch_kernel(x_ref, w1_ref, b1_ref, w2_ref, b2_ref, cls_ref, out_ref):
    xb = x_ref[0, NCLS:, :].astype(jnp.bfloat16)
    h = jax.lax.dot_general(
        xb, w1_ref[...], (((1,), (0,)), ((), ())),
        preferred_element_type=jnp.float32)
    h = _gelu(h + b1_ref[...]).astype(jnp.bfloat16)
    out = jax.lax.dot_general(
        h, w2_ref[...], (((1,), (0,)), ((), ())),
        preferred_element_type=jnp.float32) + b2_ref[...]
    out_ref[0, :NCLS, :] = cls_ref[0]
    out_ref[0, NCLS:, :] = out


def _run_local(bl, x, w1t, b1, w2t, b2, gate_pair, ain_w, ain_b, aout_w,
               aout_b):
    cls = x[:, :NCLS, :]

    def _in_map(s):
        return jnp.minimum(s, 4)

    def _out_map(s):
        return jnp.where(s < 4, 3, jnp.where(s == 4, 4, s - 5))

    cls_out = pl.pallas_call(
        functools.partial(_cls_kernel, bl),
        grid=(8,),
        in_specs=[
            pl.BlockSpec((bl, NCLS, IN_DIM), lambda s: (0, 0, 0)),
            pl.BlockSpec((NCLS, 2, IN_DIM), lambda s: (0, 0, 0)),
            pl.BlockSpec((1, HIDDEN, IN_DIM), lambda s: (_in_map(s), 0, 0)),
            pl.BlockSpec((1, 1, HIDDEN), lambda s: (_in_map(s), 0, 0)),
            pl.BlockSpec((1, IN_DIM, HIDDEN), lambda s: (_out_map(s), 0, 0)),
            pl.BlockSpec((1, 1, IN_DIM), lambda s: (_out_map(s), 0, 0)),
        ],
        out_specs=pl.BlockSpec((bl, NCLS, IN_DIM), lambda s: (0, 0, 0)),
        out_shape=jax.ShapeDtypeStruct((bl, NCLS, IN_DIM), jnp.float32),
        scratch_shapes=[
            pltpu.VMEM((12 * bl, HIDDEN), jnp.bfloat16),
            pltpu.VMEM((2, bl, NCLS), jnp.float32),
        ],
        compiler_params=pltpu.CompilerParams(
            dimension_semantics=("arbitrary",),
        ),
    )(cls, gate_pair, ain_w, ain_b, aout_w, aout_b)

    out = pl.pallas_call(
        _patch_kernel,
        grid=(bl,),
        in_specs=[
            pl.BlockSpec((1, NTOK, IN_DIM), lambda b: (b, 0, 0)),
            pl.BlockSpec((IN_DIM, HIDDEN), lambda b: (0, 0)),
            pl.BlockSpec((1, HIDDEN), lambda b: (0, 0)),
            pl.BlockSpec((HIDDEN, IN_DIM), lambda b: (0, 0)),
            pl.BlockSpec((1, IN_DIM), lambda b: (0, 0)),
            pl.BlockSpec((1, NCLS, IN_DIM), lambda b: (b, 0, 0)),
        ],
        out_specs=pl.BlockSpec((1, NTOK, IN_DIM), lambda b: (b, 0, 0)),
        out_shape=jax.ShapeDtypeStruct((bl, NTOK, IN_DIM), jnp.float32),
        compiler_params=pltpu.CompilerParams(
            dimension_semantics=("arbitrary",),
        ),
    )(x, w1t, b1, w2t, b2, cls_out)

    return out


@jax.jit
def kernel(x, patch_fc1_w, patch_fc1_b, patch_fc2_w, patch_fc2_b,
           gate_pair, atom_in_w, atom_in_b, atom_out_w, atom_out_b):
    devs = jax.devices()
    ndev = 2 if len(devs) >= 2 else 1
    mesh = Mesh(np.array(devs[:ndev]), ("d",))

    fn = jax.shard_map(
        functools.partial(_run_local, B // ndev),
        mesh=mesh,
        in_specs=(P("d"), P(), P(), P(), P(), P(), P(), P(), P(), P()),
        out_specs=P("d"),
        check_vma=False,
    )
    return fn(
        x,
        patch_fc1_w.T.astype(jnp.bfloat16), patch_fc1_b.reshape(1, HIDDEN),
        patch_fc2_w.T.astype(jnp.bfloat16), patch_fc2_b.reshape(1, IN_DIM),
        gate_pair,
        atom_in_w, atom_in_b.reshape(5, 1, HIDDEN),
        atom_out_w, atom_out_b.reshape(5, 1, IN_DIM),
    )


# speedup vs baseline: 3.1713x; 2.0131x over previous
import functools

import jax
import jax.numpy as jnp
import numpy as np
from jax.experimental import pallas as pl
from jax.experimental.pallas import tpu as pltpu

IN_DIM = 768
HIDDEN = 3072
NCLS = 6
B = 32
NPATCH = 576
NTOK = NCLS + NPATCH

SRC_TBL = np.array([[0, 3], [0, 4], [1, 3], [1, 4], [2, 3], [2, 4]], dtype=np.int32)
DST_TBL = np.array([[3, 0], [4, 0], [3, 1], [4, 1], [3, 2], [4, 2]], dtype=np.int32)

SRC_BY_ATOM = [[] for _ in range(5)]
DST_BY_ATOM = [[] for _ in range(5)]
for _n in range(NCLS):
    for _r in range(2):
        SRC_BY_ATOM[SRC_TBL[_n, _r]].append(_n * 2 + _r)
        DST_BY_ATOM[DST_TBL[_n, _r]].append(_n * 2 + _r)

IN_ATOM = [0, 1, 2, 3, 4, 4, 4, 4]
OUT_ATOM = [3, 3, 3, 3, 4, 0, 1, 2]
CAND_STEP_ATOM = {3: 3, 4: 4, 5: 0, 6: 1, 7: 2}


def _gelu(v):
    return v * (0.5 * jax.lax.erf(v * np.float32(1.0 / np.sqrt(2.0))) + 0.5)


def _cls_kernel(cls_ref, gate_ref, in_w_ref, in_b_ref, out_w_ref, out_b_ref,
                out_ref, hp_ref, gsel_ref):
    s = pl.program_id(0)

    @pl.when(s == 0)
    def _gate():
        cls = cls_ref[...]
        gw = gate_ref[...]
        cn = jnp.sqrt(jnp.sum(cls * cls, axis=-1, keepdims=True))
        gn = jnp.sqrt(jnp.sum(gw * gw, axis=-1, keepdims=True))
        gin = (cls / jnp.maximum(cn, 1e-12)).astype(jnp.bfloat16)
        gwn = (gw / jnp.maximum(gn, 1e-12)).astype(jnp.bfloat16)
        logits = jax.lax.dot_general(
            gin.reshape(B * NCLS, IN_DIM), gwn.reshape(12, IN_DIM),
            (((1,), (1,)), ((), ())),
            preferred_element_type=jnp.float32).reshape(B, NCLS, 12)
        l0 = jnp.concatenate(
            [logits[:, n, 2 * n:2 * n + 1] for n in range(NCLS)], axis=1)
        l1 = jnp.concatenate(
            [logits[:, n, 2 * n + 1:2 * n + 2] for n in range(NCLS)], axis=1)
        p0 = jax.nn.sigmoid(l0 - l1)
        pick0 = l0 >= l1
        gsel_ref[0, :, :] = jnp.where(pick0, p0, 0.0)
        gsel_ref[1, :, :] = jnp.where(pick0, 0.0, 1.0 - p0)

    @pl.when(s < 5)
    def _hidden():
        in_w = in_w_ref[0].astype(jnp.bfloat16)
        in_b = in_b_ref[0]
        for a in range(5):
            @pl.when(s == a)
            def _do(a=a):
                ks = SRC_BY_ATOM[a]
                toks = jnp.concatenate(
                    [cls_ref[:, k // 2, :] for k in ks], axis=0
                ).astype(jnp.bfloat16)
                h = jax.lax.dot_general(
                    toks, in_w, (((1,), (1,)), ((), ())),
                    preferred_element_type=jnp.float32)
                h = _gelu(h + in_b).astype(jnp.bfloat16)
                for i, k in enumerate(ks):
                    hp_ref[k * B:(k + 1) * B, :] = h[i * B:(i + 1) * B, :]

    @pl.when(s >= 3)
    def _cand():
        out_w = out_w_ref[0].astype(jnp.bfloat16)
        out_b = out_b_ref[0]
        for step, d in CAND_STEP_ATOM.items():
            @pl.when(s == step)
            def _do(step=step, d=d):
                ks = DST_BY_ATOM[d]
                h = jnp.concatenate(
                    [hp_ref[k * B:(k + 1) * B, :] for k in ks], axis=0)
                cand = jax.lax.dot_general(
                    h, out_w, (((1,), (1,)), ((), ())),
                    preferred_element_type=jnp.float32) + out_b
                for i, k in enumerate(ks):
                    n, r = k // 2, k % 2
                    w = gsel_ref[r, :, n].reshape(B, 1)
                    piece = w * cand[i * B:(i + 1) * B, :]
                    if r == 0:
                        out_ref[:, n, :] = piece
                    else:
                        out_ref[:, n, :] = out_ref[:, n, :] + piece


def _patch_kernel(x_ref, w1_ref, b1_ref, w2_ref, b2_ref, cls_ref,
                  out_ref, w1b_ref, w2b_ref):
    b = pl.program_id(0)

    @pl.when(b == 0)
    def _cast_weights():
        w1b_ref[...] = w1_ref[...].astype(jnp.bfloat16)
        w2b_ref[...] = w2_ref[...].astype(jnp.bfloat16)

    xb = x_ref[0, NCLS:, :].astype(jnp.bfloat16)
    h = jax.lax.dot_general(
        xb, w1b_ref[...], (((1,), (1,)), ((), ())),
        preferred_element_type=jnp.float32)
    h = _gelu(h + b1_ref[...]).astype(jnp.bfloat16)
    out = jax.lax.dot_general(
        h, w2b_ref[...], (((1,), (1,)), ((), ())),
        preferred_element_type=jnp.float32) + b2_ref[...]
    out_ref[0, :NCLS, :] = cls_ref[0]
    out_ref[0, NCLS:, :] = out


@jax.jit
def kernel(x, patch_fc1_w, patch_fc1_b, patch_fc2_w, patch_fc2_b,
           gate_pair, atom_in_w, atom_in_b, atom_out_w, atom_out_b):
    cls = x[:, :NCLS, :]

    def _in_map(s):
        return jnp.minimum(s, 4)

    def _out_map(s):
        return jnp.where(s < 4, 3, jnp.where(s == 4, 4, s - 5))

    cls_out = pl.pallas_call(
        _cls_kernel,
        grid=(8,),
        in_specs=[
            pl.BlockSpec((B, NCLS, IN_DIM), lambda s: (0, 0, 0)),
            pl.BlockSpec((NCLS, 2, IN_DIM), lambda s: (0, 0, 0)),
            pl.BlockSpec((1, HIDDEN, IN_DIM), lambda s: (_in_map(s), 0, 0)),
            pl.BlockSpec((1, 1, HIDDEN), lambda s: (_in_map(s), 0, 0)),
            pl.BlockSpec((1, IN_DIM, HIDDEN), lambda s: (_out_map(s), 0, 0)),
            pl.BlockSpec((1, 1, IN_DIM), lambda s: (_out_map(s), 0, 0)),
        ],
        out_specs=pl.BlockSpec((B, NCLS, IN_DIM), lambda s: (0, 0, 0)),
        out_shape=jax.ShapeDtypeStruct((B, NCLS, IN_DIM), jnp.float32),
        scratch_shapes=[
            pltpu.VMEM((12 * B, HIDDEN), jnp.bfloat16),
            pltpu.VMEM((2, B, NCLS), jnp.float32),
        ],
        compiler_params=pltpu.CompilerParams(
            dimension_semantics=("arbitrary",),
        ),
    )(cls, gate_pair, atom_in_w, atom_in_b.reshape(5, 1, HIDDEN),
      atom_out_w, atom_out_b.reshape(5, 1, IN_DIM))

    out = pl.pallas_call(
        _patch_kernel,
        grid=(B,),
        in_specs=[
            pl.BlockSpec((1, NTOK, IN_DIM), lambda b: (b, 0, 0)),
            pl.BlockSpec((HIDDEN, IN_DIM), lambda b: (0, 0)),
            pl.BlockSpec((1, HIDDEN), lambda b: (0, 0)),
            pl.BlockSpec((IN_DIM, HIDDEN), lambda b: (0, 0)),
            pl.BlockSpec((1, IN_DIM), lambda b: (0, 0)),
            pl.BlockSpec((1, NCLS, IN_DIM), lambda b: (b, 0, 0)),
        ],
        out_specs=pl.BlockSpec((1, NTOK, IN_DIM), lambda b: (b, 0, 0)),
        out_shape=jax.ShapeDtypeStruct((B, NTOK, IN_DIM), jnp.float32),
        scratch_shapes=[
            pltpu.VMEM((HIDDEN, IN_DIM), jnp.bfloat16),
            pltpu.VMEM((IN_DIM, HIDDEN), jnp.bfloat16),
        ],
        compiler_params=pltpu.CompilerParams(
            dimension_semantics=("arbitrary",),
        ),
    )(x, patch_fc1_w, patch_fc1_b.reshape(1, HIDDEN), patch_fc2_w,
      patch_fc2_b.reshape(1, IN_DIM), cls_out)

    return out


# speedup vs baseline: 3.2290x; 1.0182x over previous
import functools

import jax
import jax.numpy as jnp
import numpy as np
from jax.experimental import pallas as pl
from jax.experimental.pallas import tpu as pltpu

IN_DIM = 768
HIDDEN = 3072
NCLS = 6
B = 32
NPATCH = 576
NTOK = NCLS + NPATCH

SRC_TBL = np.array([[0, 3], [0, 4], [1, 3], [1, 4], [2, 3], [2, 4]], dtype=np.int32)
DST_TBL = np.array([[3, 0], [4, 0], [3, 1], [4, 1], [3, 2], [4, 2]], dtype=np.int32)

SRC_BY_ATOM = [[] for _ in range(5)]
DST_BY_ATOM = [[] for _ in range(5)]
for _n in range(NCLS):
    for _r in range(2):
        SRC_BY_ATOM[SRC_TBL[_n, _r]].append(_n * 2 + _r)
        DST_BY_ATOM[DST_TBL[_n, _r]].append(_n * 2 + _r)

IN_ATOM = [0, 1, 2, 3, 4, 4, 4, 4]
OUT_ATOM = [3, 3, 3, 3, 4, 0, 1, 2]
CAND_STEP_ATOM = {3: 3, 4: 4, 5: 0, 6: 1, 7: 2}


def _gelu(v):
    return v * (0.5 * jax.lax.erf(v * np.float32(1.0 / np.sqrt(2.0))) + 0.5)


def _cls_kernel(cls_ref, gate_ref, in_w_ref, in_b_ref, out_w_ref, out_b_ref,
                out_ref, hp_ref, gsel_ref):
    s = pl.program_id(0)

    @pl.when(s == 0)
    def _gate():
        cls = cls_ref[...]
        gw = gate_ref[...]
        cn = jnp.sqrt(jnp.sum(cls * cls, axis=-1, keepdims=True))
        gn = jnp.sqrt(jnp.sum(gw * gw, axis=-1, keepdims=True))
        gin = (cls / jnp.maximum(cn, 1e-12)).astype(jnp.bfloat16)
        gwn = (gw / jnp.maximum(gn, 1e-12)).astype(jnp.bfloat16)
        logits = jax.lax.dot_general(
            gin.reshape(B * NCLS, IN_DIM), gwn.reshape(12, IN_DIM),
            (((1,), (1,)), ((), ())),
            preferred_element_type=jnp.float32).reshape(B, NCLS, 12)
        l0 = jnp.concatenate(
            [logits[:, n, 2 * n:2 * n + 1] for n in range(NCLS)], axis=1)
        l1 = jnp.concatenate(
            [logits[:, n, 2 * n + 1:2 * n + 2] for n in range(NCLS)], axis=1)
        p0 = jax.nn.sigmoid(l0 - l1)
        pick0 = l0 >= l1
        gsel_ref[0, :, :] = jnp.where(pick0, p0, 0.0)
        gsel_ref[1, :, :] = jnp.where(pick0, 0.0, 1.0 - p0)

    @pl.when(s < 5)
    def _hidden():
        in_w = in_w_ref[0].astype(jnp.bfloat16)
        in_b = in_b_ref[0]
        for a in range(5):
            @pl.when(s == a)
            def _do(a=a):
                ks = SRC_BY_ATOM[a]
                toks = jnp.concatenate(
                    [cls_ref[:, k // 2, :] for k in ks], axis=0
                ).astype(jnp.bfloat16)
                h = jax.lax.dot_general(
                    toks, in_w, (((1,), (1,)), ((), ())),
                    preferred_element_type=jnp.float32)
                h = _gelu(h + in_b).astype(jnp.bfloat16)
                for i, k in enumerate(ks):
                    hp_ref[k * B:(k + 1) * B, :] = h[i * B:(i + 1) * B, :]

    @pl.when(s >= 3)
    def _cand():
        out_w = out_w_ref[0].astype(jnp.bfloat16)
        out_b = out_b_ref[0]
        for step, d in CAND_STEP_ATOM.items():
            @pl.when(s == step)
            def _do(step=step, d=d):
                ks = DST_BY_ATOM[d]
                h = jnp.concatenate(
                    [hp_ref[k * B:(k + 1) * B, :] for k in ks], axis=0)
                cand = jax.lax.dot_general(
                    h, out_w, (((1,), (1,)), ((), ())),
                    preferred_element_type=jnp.float32) + out_b
                for i, k in enumerate(ks):
                    n, r = k // 2, k % 2
                    w = gsel_ref[r, :, n].reshape(B, 1)
                    piece = w * cand[i * B:(i + 1) * B, :]
                    if r == 0:
                        out_ref[:, n, :] = piece
                    else:
                        out_ref[:, n, :] = out_ref[:, n, :] + piece


BPS = 2


def _patch_kernel(x_ref, w1_ref, b1_ref, w2_ref, b2_ref, cls_ref,
                  out_ref, w1b_ref, w2b_ref):
    b = pl.program_id(0)

    @pl.when(b == 0)
    def _cast_weights():
        w1b_ref[...] = w1_ref[...].astype(jnp.bfloat16)
        w2b_ref[...] = w2_ref[...].astype(jnp.bfloat16)

    xb = jnp.concatenate(
        [x_ref[j, NCLS:, :] for j in range(BPS)], axis=0
    ).astype(jnp.bfloat16)
    h = jax.lax.dot_general(
        xb, w1b_ref[...], (((1,), (1,)), ((), ())),
        preferred_element_type=jnp.float32)
    h = _gelu(h + b1_ref[...]).astype(jnp.bfloat16)
    out = jax.lax.dot_general(
        h, w2b_ref[...], (((1,), (1,)), ((), ())),
        preferred_element_type=jnp.float32) + b2_ref[...]
    for j in range(BPS):
        out_ref[j, :NCLS, :] = cls_ref[j]
        out_ref[j, NCLS:, :] = out[j * NPATCH:(j + 1) * NPATCH, :]


@jax.jit
def kernel(x, patch_fc1_w, patch_fc1_b, patch_fc2_w, patch_fc2_b,
           gate_pair, atom_in_w, atom_in_b, atom_out_w, atom_out_b):
    cls = x[:, :NCLS, :]

    def _in_map(s):
        return jnp.minimum(s, 4)

    def _out_map(s):
        return jnp.where(s < 4, 3, jnp.where(s == 4, 4, s - 5))

    cls_out = pl.pallas_call(
        _cls_kernel,
        grid=(8,),
        in_specs=[
            pl.BlockSpec((B, NCLS, IN_DIM), lambda s: (0, 0, 0)),
            pl.BlockSpec((NCLS, 2, IN_DIM), lambda s: (0, 0, 0)),
            pl.BlockSpec((1, HIDDEN, IN_DIM), lambda s: (_in_map(s), 0, 0)),
            pl.BlockSpec((1, 1, HIDDEN), lambda s: (_in_map(s), 0, 0)),
            pl.BlockSpec((1, IN_DIM, HIDDEN), lambda s: (_out_map(s), 0, 0)),
            pl.BlockSpec((1, 1, IN_DIM), lambda s: (_out_map(s), 0, 0)),
        ],
        out_specs=pl.BlockSpec((B, NCLS, IN_DIM), lambda s: (0, 0, 0)),
        out_shape=jax.ShapeDtypeStruct((B, NCLS, IN_DIM), jnp.float32),
        scratch_shapes=[
            pltpu.VMEM((12 * B, HIDDEN), jnp.bfloat16),
            pltpu.VMEM((2, B, NCLS), jnp.float32),
        ],
        compiler_params=pltpu.CompilerParams(
            dimension_semantics=("arbitrary",),
        ),
    )(cls, gate_pair, atom_in_w, atom_in_b.reshape(5, 1, HIDDEN),
      atom_out_w, atom_out_b.reshape(5, 1, IN_DIM))

    out = pl.pallas_call(
        _patch_kernel,
        grid=(B // BPS,),
        in_specs=[
            pl.BlockSpec((BPS, NTOK, IN_DIM), lambda b: (b, 0, 0)),
            pl.BlockSpec((HIDDEN, IN_DIM), lambda b: (0, 0)),
            pl.BlockSpec((1, HIDDEN), lambda b: (0, 0)),
            pl.BlockSpec((IN_DIM, HIDDEN), lambda b: (0, 0)),
            pl.BlockSpec((1, IN_DIM), lambda b: (0, 0)),
            pl.BlockSpec((BPS, NCLS, IN_DIM), lambda b: (b, 0, 0)),
        ],
        out_specs=pl.BlockSpec((BPS, NTOK, IN_DIM), lambda b: (b, 0, 0)),
        out_shape=jax.ShapeDtypeStruct((B, NTOK, IN_DIM), jnp.float32),
        scratch_shapes=[
            pltpu.VMEM((HIDDEN, IN_DIM), jnp.bfloat16),
            pltpu.VMEM((IN_DIM, HIDDEN), jnp.bfloat16),
        ],
        compiler_params=pltpu.CompilerParams(
            dimension_semantics=("arbitrary",),
        ),
    )(x, patch_fc1_w, patch_fc1_b.reshape(1, HIDDEN), patch_fc2_w,
      patch_fc2_b.reshape(1, IN_DIM), cls_out)

    return out
